# Initial kernel scaffold; baseline (speedup 1.0000x reference)
import jax
import jax.numpy as jnp
from jax.experimental import pallas as pl


def kernel(xyz1, xyz2, points1, points2, vel1, vel2, mask1, mask2, generator, w_xyz, w_vel, w_points, mlp_w0, mlp_b0, mlp_w1, mlp_b1, wn1_w0, wn1_b0, wn1_w1, wn1_b1, wn1_w2, wn1_b2, wn2_w0, wn2_b0, wn2_w1, wn2_b1, wn2_w2, wn2_b2):
    raise NotImplementedError("write your pallas kernel here")


# speedup vs baseline: 2.2617x; 2.2617x over previous
import functools

import jax
import jax.numpy as jnp
from jax import lax
from jax.experimental import pallas as pl

NSAMPLE = 16
MIN_COUNT = 8
N = 4096
BQ = 256
FPAD = 128


def _knn_body(k, q_ref, db_ref, out_ref):
    q = q_ref[0]
    db = db_ref[0]
    qn = jnp.sum(q * q, axis=1, keepdims=True)
    dn = jnp.sum(db * db, axis=1, keepdims=True).T
    d2 = lax.dot_general(q, db, (((1,), (1,)), ((), ())),
                         preferred_element_type=jnp.float32,
                         precision=lax.Precision.DEFAULT)
    dist = jnp.maximum(-2.0 * d2 + qn + dn, 0.0)
    nd = -dist
    iota = lax.broadcasted_iota(jnp.int32, (BQ, N), 1)
    cols = []
    for _ in range(k):
        m = jnp.max(nd, axis=1, keepdims=True)
        sel = jnp.where(nd == m, iota, N)
        idx = jnp.min(sel, axis=1, keepdims=True)
        cols.append(idx)
        nd = jnp.where(iota == idx, -jnp.inf, nd)
    cols.append(jnp.zeros((BQ, FPAD - k), jnp.int32))
    out_ref[0] = jnp.concatenate(cols, axis=1)


def _knn_topk(q, db, k):
    b = q.shape[0]
    out = pl.pallas_call(
        functools.partial(_knn_body, k),
        grid=(b, N // BQ),
        in_specs=[
            pl.BlockSpec((1, BQ, FPAD), lambda bi, i: (bi, i, 0)),
            pl.BlockSpec((1, N, FPAD), lambda bi, i: (bi, 0, 0)),
        ],
        out_specs=pl.BlockSpec((1, BQ, FPAD), lambda bi, i: (bi, i, 0)),
        out_shape=jax.ShapeDtypeStruct((b, N, FPAD), jnp.int32),
    )(q, db)
    return out[:, :, :k]


def _pad_last(x, width):
    return jnp.pad(x, [(0, 0)] * (x.ndim - 1) + [(0, width - x.shape[-1])])


def kernel(xyz1, xyz2, points1, points2, vel1, vel2, mask1, mask2, generator,
           w_xyz, w_vel, w_points, mlp_w0, mlp_b0, mlp_w1, mlp_b1,
           wn1_w0, wn1_b0, wn1_w1, wn1_b1, wn1_w2, wn1_b2,
           wn2_w0, wn2_b0, wn2_w1, wn2_b1, wn2_w2, wn2_b2):
    B = xyz1.shape[0]
    x1 = jnp.swapaxes(xyz1, 1, 2)
    x2 = jnp.swapaxes(xyz2, 1, 2)
    p1 = jnp.swapaxes(points1, 1, 2)
    p2 = jnp.swapaxes(points2, 1, 2)

    x1p = _pad_last(x1, FPAD)
    x2p = _pad_last(x2, FPAD)
    cidx = _knn_topk(x1p, x2p, MIN_COUNT)

    f1 = _pad_last(jnp.concatenate([w_xyz * x1, w_points * p1], axis=-1), FPAD)
    f2 = _pad_last(jnp.concatenate([w_xyz * x2, w_points * p2], axis=-1), FPAD)
    kidx = _knn_topk(f1, f2, NSAMPLE)

    gather = jax.vmap(lambda p, i: p[i])
    ccoords = gather(x2, cidx)
    cvel = gather(vel2[:, :, None], cidx)[..., 0]
    u = ccoords / jnp.linalg.norm(ccoords, axis=-1, keepdims=True)
    ATA = jnp.matmul(jnp.swapaxes(u, -1, -2), u) + 1e-06 * jnp.eye(3, dtype=u.dtype)
    ATb = jnp.matmul(jnp.swapaxes(u, -1, -2), cvel[..., None])
    rigid = jnp.linalg.solve(ATA, ATb)[..., 0]

    rp = _pad_last(rigid, FPAD)
    kidx2 = _knn_topk(rp, rp, NSAMPLE)

    nxyz = gather(x2, kidx)
    direction = nxyz - x1[:, :, None, :]
    g2 = gather(p2, kidx)
    g1 = jnp.broadcast_to(p1[:, :, None, :], (B, N, NSAMPLE, 64))
    new_points = jnp.concatenate([g1, g2, direction], axis=-1)
    h = jnp.einsum('oc,bnkc->bnko', mlp_w0, new_points) + mlp_b0
    h = jax.nn.leaky_relu(h, 0.1)
    h = jnp.einsum('oc,bnkc->bnko', mlp_w1, h) + mlp_b1
    h = jax.nn.leaky_relu(h, 0.1)

    w = direction
    for (ww, bb) in [(wn1_w0, wn1_b0), (wn1_w1, wn1_b1), (wn1_w2, wn1_b2)]:
        w = jax.nn.relu(jnp.einsum('oc,bnkc->bnko', ww, w) + bb)
    p2p = jnp.sum(w * h, axis=2)

    nxyz2 = gather(x1, kidx2)
    dir2 = nxyz2 - x1[:, :, None, :]
    w2 = dir2
    for (ww, bb) in [(wn2_w0, wn2_b0), (wn2_w1, wn2_b1), (wn2_w2, wn2_b2)]:
        w2 = jax.nn.relu(jnp.einsum('oc,bnkc->bnko', ww, w2) + bb)
    gc = gather(p2p, kidx2)
    patch = jnp.sum(w2 * gc, axis=2)

    return (jnp.swapaxes(patch, 1, 2), rigid)


# speedup vs baseline: 17.3671x; 7.6789x over previous
import functools

import jax
import jax.numpy as jnp
from jax import lax
from jax.experimental import pallas as pl
from jax.experimental.pallas import tpu as pltpu
from jax.experimental.pallas import tpu_sc as plsc

NSAMPLE = 16
MIN_COUNT = 8
N = 4096
BQ = 256
BR = 256
FPAD = 128
TW = 128


def _knn_body(k, q_ref, db_ref, out_ref):
    q = q_ref[0]
    db = db_ref[0]
    qn = jnp.sum(q * q, axis=1, keepdims=True)
    dn = jnp.sum(db * db, axis=1, keepdims=True).T
    d2 = lax.dot_general(q, db, (((1,), (1,)), ((), ())),
                         preferred_element_type=jnp.float32,
                         precision=lax.Precision.DEFAULT)
    dist = jnp.maximum(-2.0 * d2 + qn + dn, 0.0)
    nd = -dist
    iota = lax.broadcasted_iota(jnp.int32, (BQ, N), 1)
    cols = []
    for _ in range(k):
        m = jnp.max(nd, axis=1, keepdims=True)
        sel = jnp.where(nd == m, iota, N)
        idx = jnp.min(sel, axis=1, keepdims=True)
        cols.append(idx)
        nd = jnp.where(iota == idx, -jnp.inf, nd)
    cols.append(jnp.zeros((BQ, FPAD - k), jnp.int32))
    out_ref[0] = jnp.concatenate(cols, axis=1)


def _knn_topk(q, db, k):
    b = q.shape[0]
    out = pl.pallas_call(
        functools.partial(_knn_body, k),
        grid=(b, N // BQ),
        in_specs=[
            pl.BlockSpec((1, BQ, FPAD), lambda bi, i: (bi, i, 0)),
            pl.BlockSpec((1, N, FPAD), lambda bi, i: (bi, 0, 0)),
        ],
        out_specs=pl.BlockSpec((1, BQ, FPAD), lambda bi, i: (bi, i, 0)),
        out_shape=jax.ShapeDtypeStruct((b, N, FPAD), jnp.int32),
    )(q, db)
    return out[:, :, :k]


def _sc_gather(table, idx):
    _, d = table.shape
    (m,) = idx.shape
    info = plsc.get_sparse_core_info()
    nw = info.num_cores * info.num_subcores
    m_per_w = m // nw
    ch = min(m_per_w, 256)
    n_ch = m_per_w // ch
    mesh = plsc.VectorSubcoreMesh(core_axis_name="c", subcore_axis_name="s")

    @functools.partial(
        pl.kernel, mesh=mesh,
        out_type=jax.ShapeDtypeStruct((m, d), jnp.float32),
        scratch_types=[
            pltpu.VMEM((m_per_w,), jnp.int32),
            pltpu.VMEM((ch, d), jnp.float32),
            pltpu.SemaphoreType.DMA,
        ],
    )
    def gk(table_hbm, idx_hbm, out_hbm, idx_v, rows_v, sem):
        wid = lax.axis_index("s") * info.num_cores + lax.axis_index("c")
        base = wid * m_per_w
        pltpu.sync_copy(idx_hbm.at[pl.ds(base, m_per_w)], idx_v)
        for c in range(n_ch):
            pltpu.async_copy(
                table_hbm.at[idx_v.at[pl.ds(c * ch, ch)]], rows_v, sem).wait()
            pltpu.sync_copy(rows_v, out_hbm.at[pl.ds(base + c * ch, ch)])

    return gk(table, idx)


def _prep_body(p1_ref, p2_ref, x2_ref, w1a_ref, w1b_ref, b1_ref,
               a_ref, t_ref):
    p1 = p1_ref[...]
    p2 = p2_ref[...]
    a_ref[...] = lax.dot_general(p1, w1a_ref[...], (((1,), (0,)), ((), ())),
                                 preferred_element_type=jnp.float32) + b1_ref[...]
    t64 = lax.dot_general(p2, w1b_ref[...], (((1,), (0,)), ((), ())),
                          preferred_element_type=jnp.float32)
    x2 = x2_ref[...][:, :3]
    t_ref[...] = jnp.concatenate(
        [t64, x2, jnp.zeros((t64.shape[0], TW - 67), jnp.float32)], axis=1)


def _prep(p1f, p2f, x2f, w1a, w1b, b1):
    rows = p1f.shape[0]
    return pl.pallas_call(
        _prep_body,
        grid=(rows // BR,),
        in_specs=[
            pl.BlockSpec((BR, 64), lambda i: (i, 0)),
            pl.BlockSpec((BR, 64), lambda i: (i, 0)),
            pl.BlockSpec((BR, FPAD), lambda i: (i, 0)),
            pl.BlockSpec((64, 64), lambda i: (0, 0)),
            pl.BlockSpec((64, 64), lambda i: (0, 0)),
            pl.BlockSpec((1, 64), lambda i: (0, 0)),
        ],
        out_specs=[
            pl.BlockSpec((BR, 64), lambda i: (i, 0)),
            pl.BlockSpec((BR, TW), lambda i: (i, 0)),
        ],
        out_shape=[
            jax.ShapeDtypeStruct((rows, 64), jnp.float32),
            jax.ShapeDtypeStruct((rows, TW), jnp.float32),
        ],
    )(p1f, p2f, x2f, w1a, w1b, b1)


def _bf16(x):
    return x.astype(jnp.bfloat16).astype(jnp.float32)


def _rigid_body(g_ref, out_ref):
    g = g_ref[...]
    axx = ayy = azz = axy = axz = ayz = 0.0
    bx = by = bz = 0.0
    for k in range(MIN_COUNT):
        o = k * TW
        cx = g[:, o:o + 1]
        cy = g[:, o + 1:o + 2]
        cz = g[:, o + 2:o + 3]
        v = g[:, o + 3:o + 4]
        nrm = jnp.sqrt(cx * cx + cy * cy + cz * cz)
        ux, uy, uz = _bf16(cx / nrm), _bf16(cy / nrm), _bf16(cz / nrm)
        vb = _bf16(v)
        axx += ux * ux
        ayy += uy * uy
        azz += uz * uz
        axy += ux * uy
        axz += ux * uz
        ayz += uy * uz
        bx += ux * vb
        by += uy * vb
        bz += uz * vb
    axx += 1e-06
    ayy += 1e-06
    azz += 1e-06
    rows = [[axx, axy, axz, bx], [axy, ayy, ayz, by], [axz, ayz, azz, bz]]
    m0, m1, m2 = jnp.abs(axx), jnp.abs(axy), jnp.abs(axz)
    c0 = (m0 >= m1) & (m0 >= m2)
    c1 = (~c0) & (m1 >= m2)

    def pick(v):
        return jnp.where(c0, v[0], jnp.where(c1, v[1], v[2]))

    r0 = [pick([rows[0][j], rows[1][j], rows[2][j]]) for j in range(4)]
    r1 = [pick([rows[1][j], rows[0][j], rows[0][j]]) for j in range(4)]
    r2 = [pick([rows[2][j], rows[2][j], rows[1][j]]) for j in range(4)]
    l10 = r1[0] / r0[0]
    l20 = r2[0] / r0[0]
    r1 = [r1[j] - l10 * r0[j] for j in range(4)]
    r2 = [r2[j] - l20 * r0[j] for j in range(4)]
    swap = jnp.abs(r2[1]) > jnp.abs(r1[1])
    s1 = [jnp.where(swap, r2[j], r1[j]) for j in range(4)]
    s2 = [jnp.where(swap, r1[j], r2[j]) for j in range(4)]
    l21 = s2[1] / s1[1]
    s2 = [s2[j] - l21 * s1[j] for j in range(4)]
    rz = s2[3] / s2[2]
    ry = (s1[3] - s1[2] * rz) / s1[1]
    rx = (r0[3] - r0[1] * ry - r0[2] * rz) / r0[0]
    out_ref[...] = jnp.concatenate(
        [rx, ry, rz, jnp.zeros((rx.shape[0], FPAD - 3), jnp.float32)], axis=1)


def _rigid(g0):
    rows = g0.shape[0]
    return pl.pallas_call(
        _rigid_body,
        grid=(rows // BR,),
        in_specs=[pl.BlockSpec((BR, MIN_COUNT * TW), lambda i: (i, 0))],
        out_specs=pl.BlockSpec((BR, FPAD), lambda i: (i, 0)),
        out_shape=jax.ShapeDtypeStruct((rows, FPAD), jnp.float32),
    )(g0)


def _dot(a, b):
    return lax.dot_general(a, b, (((1,), (0,)), ((), ())),
                           preferred_element_type=jnp.float32,
                           precision=lax.Precision.DEFAULT)


def _leaky(x):
    return jnp.where(x >= 0, x, 0.1 * x)


def _wnet(d, wts):
    (v1, c1), (v2, c2), (v3, c3) = wts
    l1 = jnp.maximum(_dot(d, v1) + c1, 0.0)
    l2 = jnp.maximum(_dot(l1, v2) + c2, 0.0)
    return jnp.maximum(_dot(l2, v3) + c3, 0.0)


def _rep_rows(x, k):
    r, c = x.shape
    return jnp.reshape(jnp.broadcast_to(x[:, None, :], (r, k, c)), (r * k, c))


def _p2p_body(g_ref, a_ref, x1_ref, w1c_ref, w2_ref, b2_ref,
              v1_ref, c1_ref, v2_ref, c2_ref, v3_ref, c3_ref, out_ref):
    g = g_ref[...]
    a = _rep_rows(a_ref[...], NSAMPLE)
    x1 = _rep_rows(x1_ref[...][:, :3], NSAMPLE)
    d = g[:, 64:67] - x1
    pre = a + g[:, :64] + _dot(d, w1c_ref[...])
    h = _leaky(pre)
    h2 = _leaky(_dot(h, w2_ref[...]) + b2_ref[...])
    w = _wnet(d, [(v1_ref[...], c1_ref[...]), (v2_ref[...], c2_ref[...]),
                  (v3_ref[...], c3_ref[...])])
    prod = w * h2
    out_ref[...] = jnp.sum(
        jnp.reshape(prod, (BR, NSAMPLE, 64)), axis=1)


def _p2p_stage(g, a, x1f, w1c, w2, b2, wn):
    rows = a.shape[0]
    (v1, c1), (v2, c2), (v3, c3) = wn
    return pl.pallas_call(
        _p2p_body,
        grid=(rows // BR,),
        in_specs=[
            pl.BlockSpec((BR * NSAMPLE, TW), lambda i: (i, 0)),
            pl.BlockSpec((BR, 64), lambda i: (i, 0)),
            pl.BlockSpec((BR, FPAD), lambda i: (i, 0)),
            pl.BlockSpec((3, 64), lambda i: (0, 0)),
            pl.BlockSpec((64, 64), lambda i: (0, 0)),
            pl.BlockSpec((1, 64), lambda i: (0, 0)),
            pl.BlockSpec((3, 8), lambda i: (0, 0)),
            pl.BlockSpec((1, 8), lambda i: (0, 0)),
            pl.BlockSpec((8, 8), lambda i: (0, 0)),
            pl.BlockSpec((1, 8), lambda i: (0, 0)),
            pl.BlockSpec((8, 64), lambda i: (0, 0)),
            pl.BlockSpec((1, 64), lambda i: (0, 0)),
        ],
        out_specs=pl.BlockSpec((BR, 64), lambda i: (i, 0)),
        out_shape=jax.ShapeDtypeStruct((rows, 64), jnp.float32),
    )(g, a, x1f, w1c, w2, b2, v1, c1, v2, c2, v3, c3)


def _patch_body(g_ref, x1_ref, u1_ref, e1_ref, u2_ref, e2_ref,
                u3_ref, e3_ref, out_ref):
    g = g_ref[...]
    x1 = _rep_rows(x1_ref[...][:, :3], NSAMPLE)
    d = g[:, 64:67] - x1
    w = _wnet(d, [(u1_ref[...], e1_ref[...]), (u2_ref[...], e2_ref[...]),
                  (u3_ref[...], e3_ref[...])])
    prod = w * g[:, :64]
    out_ref[...] = jnp.sum(
        jnp.reshape(prod, (BR, NSAMPLE, 64)), axis=1)


def _patch_stage(g, x1f, wn):
    rows = x1f.shape[0]
    (u1, e1), (u2, e2), (u3, e3) = wn
    return pl.pallas_call(
        _patch_body,
        grid=(rows // BR,),
        in_specs=[
            pl.BlockSpec((BR * NSAMPLE, TW), lambda i: (i, 0)),
            pl.BlockSpec((BR, FPAD), lambda i: (i, 0)),
            pl.BlockSpec((3, 8), lambda i: (0, 0)),
            pl.BlockSpec((1, 8), lambda i: (0, 0)),
            pl.BlockSpec((8, 8), lambda i: (0, 0)),
            pl.BlockSpec((1, 8), lambda i: (0, 0)),
            pl.BlockSpec((8, 64), lambda i: (0, 0)),
            pl.BlockSpec((1, 64), lambda i: (0, 0)),
        ],
        out_specs=pl.BlockSpec((BR, 64), lambda i: (i, 0)),
        out_shape=jax.ShapeDtypeStruct((rows, 64), jnp.float32),
    )(g, x1f, u1, e1, u2, e2, u3, e3)


def _pad_last(x, width):
    return jnp.pad(x, [(0, 0)] * (x.ndim - 1) + [(0, width - x.shape[-1])])


def kernel(xyz1, xyz2, points1, points2, vel1, vel2, mask1, mask2, generator,
           w_xyz, w_vel, w_points, mlp_w0, mlp_b0, mlp_w1, mlp_b1,
           wn1_w0, wn1_b0, wn1_w1, wn1_b1, wn1_w2, wn1_b2,
           wn2_w0, wn2_b0, wn2_w1, wn2_b1, wn2_w2, wn2_b2):
    B = xyz1.shape[0]
    x1 = jnp.swapaxes(xyz1, 1, 2)
    x2 = jnp.swapaxes(xyz2, 1, 2)
    p1 = jnp.swapaxes(points1, 1, 2)
    p2 = jnp.swapaxes(points2, 1, 2)

    x1p = _pad_last(x1, FPAD)
    x2p = _pad_last(x2, FPAD)
    x1f = jnp.reshape(x1p, (B * N, FPAD))
    x2f = jnp.reshape(x2p, (B * N, FPAD))
    p1f = jnp.reshape(p1, (B * N, 64))
    p2f = jnp.reshape(p2, (B * N, 64))
    roff = (jnp.arange(B, dtype=jnp.int32) * N)[:, None, None]

    cidx = _knn_topk(x1p, x2p, MIN_COUNT)
    cflat = jnp.reshape(cidx + roff, (B * N * MIN_COUNT,))

    f1 = _pad_last(jnp.concatenate([w_xyz * x1, w_points * p1], axis=-1), FPAD)
    f2 = _pad_last(jnp.concatenate([w_xyz * x2, w_points * p2], axis=-1), FPAD)
    kidx = _knn_topk(f1, f2, NSAMPLE)
    kflat = jnp.reshape(kidx + roff, (B * N * NSAMPLE,))

    tab0 = _pad_last(
        jnp.concatenate([x2f[:, :3], jnp.reshape(vel2, (B * N, 1))], axis=1), TW)
    g0 = _sc_gather(tab0, cflat)
    rigidp = _rigid(jnp.reshape(g0, (B * N, MIN_COUNT * TW)))
    rigid = jnp.reshape(rigidp[:, :3], (B, N, 3))

    kidx2 = _knn_topk(jnp.reshape(rigidp, (B, N, FPAD)),
                      jnp.reshape(rigidp, (B, N, FPAD)), NSAMPLE)
    k2flat = jnp.reshape(kidx2 + roff, (B * N * NSAMPLE,))

    w1a = jnp.transpose(mlp_w0[:, :64])
    w1b = jnp.transpose(mlp_w0[:, 64:128])
    w1c = jnp.transpose(mlp_w0[:, 128:131])
    a, t = _prep(p1f, p2f, x2f, w1a, w1b, mlp_b0[None, :])

    g1 = _sc_gather(t, kflat)
    wn1 = [(jnp.transpose(wn1_w0), wn1_b0[None, :]),
           (jnp.transpose(wn1_w1), wn1_b1[None, :]),
           (jnp.transpose(wn1_w2), wn1_b2[None, :])]
    p2p = _p2p_stage(g1, a, x1f, w1c, jnp.transpose(mlp_w1),
                     mlp_b1[None, :], wn1)

    tab2 = _pad_last(jnp.concatenate([p2p, x1f[:, :3]], axis=1), TW)
    g2 = _sc_gather(tab2, k2flat)
    wn2 = [(jnp.transpose(wn2_w0), wn2_b0[None, :]),
           (jnp.transpose(wn2_w1), wn2_b1[None, :]),
           (jnp.transpose(wn2_w2), wn2_b2[None, :])]
    patch = _patch_stage(g2, x1f, wn2)

    patch = jnp.transpose(jnp.reshape(patch, (B, N, 64)), (0, 2, 1))
    return (patch, rigid)


# speedup vs baseline: 20.4504x; 1.1775x over previous
import functools

import jax
import jax.numpy as jnp
from jax import lax
from jax.experimental import pallas as pl
from jax.experimental.pallas import tpu as pltpu
from jax.experimental.pallas import tpu_sc as plsc

NSAMPLE = 16
MIN_COUNT = 8
N = 4096
BQ = 256
BR = 256
FPAD = 128
TW = 128


def _knn_body(k, q_ref, db_ref, out_ref):
    q = q_ref[0]
    db = db_ref[0]
    qn = jnp.sum(q * q, axis=1, keepdims=True)
    dn = jnp.sum(db * db, axis=1, keepdims=True).T
    d2 = lax.dot_general(q, db, (((1,), (1,)), ((), ())),
                         preferred_element_type=jnp.float32,
                         precision=lax.Precision.DEFAULT)
    nd = jnp.minimum((2.0 * d2 - qn) - dn, 0.0)
    iota = lax.broadcasted_iota(jnp.int32, (BQ, N), 1).astype(jnp.float32)
    cols = []
    idx = None
    for j in range(k):
        if j > 0:
            nd = jnp.where(iota == idx, -jnp.inf, nd)
        m = jnp.max(nd, axis=1, keepdims=True)
        sel = jnp.where(nd == m, iota, float(N))
        idx = jnp.min(sel, axis=1, keepdims=True)
        cols.append(idx)
    icols = jnp.concatenate(cols, axis=1).astype(jnp.int32)
    out_ref[0] = jnp.concatenate(
        [icols, jnp.zeros((BQ, FPAD - k), jnp.int32)], axis=1)


def _knn_topk(q, db, k):
    b = q.shape[0]
    out = pl.pallas_call(
        functools.partial(_knn_body, k),
        grid=(b, N // BQ),
        in_specs=[
            pl.BlockSpec((1, BQ, FPAD), lambda bi, i: (bi, i, 0)),
            pl.BlockSpec((1, N, FPAD), lambda bi, i: (bi, 0, 0)),
        ],
        out_specs=pl.BlockSpec((1, BQ, FPAD), lambda bi, i: (bi, i, 0)),
        out_shape=jax.ShapeDtypeStruct((b, N, FPAD), jnp.int32),
    )(q, db)
    return out[:, :, :k]


def _sc_gather(table, idx):
    _, d = table.shape
    (m,) = idx.shape
    info = plsc.get_sparse_core_info()
    nw = info.num_cores * info.num_subcores
    m_per_w = m // nw
    ch = min(m_per_w, 256)
    n_ch = m_per_w // ch
    mesh = plsc.VectorSubcoreMesh(core_axis_name="c", subcore_axis_name="s")

    @functools.partial(
        pl.kernel, mesh=mesh,
        out_type=jax.ShapeDtypeStruct((m, d), jnp.float32),
        scratch_types=[
            pltpu.VMEM((m_per_w,), jnp.int32),
            pltpu.VMEM((ch, d), jnp.float32),
            pltpu.SemaphoreType.DMA,
        ],
    )
    def gk(table_hbm, idx_hbm, out_hbm, idx_v, rows_v, sem):
        wid = lax.axis_index("s") * info.num_cores + lax.axis_index("c")
        base = wid * m_per_w
        pltpu.sync_copy(idx_hbm.at[pl.ds(base, m_per_w)], idx_v)
        for c in range(n_ch):
            pltpu.async_copy(
                table_hbm.at[idx_v.at[pl.ds(c * ch, ch)]], rows_v, sem).wait()
            pltpu.sync_copy(rows_v, out_hbm.at[pl.ds(base + c * ch, ch)])

    return gk(table, idx)


def _prep_body(p1_ref, p2_ref, x2_ref, w1a_ref, w1b_ref, b1_ref,
               a_ref, t_ref):
    p1 = p1_ref[...]
    p2 = p2_ref[...]
    a_ref[...] = lax.dot_general(p1, w1a_ref[...], (((1,), (0,)), ((), ())),
                                 preferred_element_type=jnp.float32) + b1_ref[...]
    t64 = lax.dot_general(p2, w1b_ref[...], (((1,), (0,)), ((), ())),
                          preferred_element_type=jnp.float32)
    x2 = x2_ref[...][:, :3]
    t_ref[...] = jnp.concatenate(
        [t64, x2, jnp.zeros((t64.shape[0], TW - 67), jnp.float32)], axis=1)


def _prep(p1f, p2f, x2f, w1a, w1b, b1):
    rows = p1f.shape[0]
    return pl.pallas_call(
        _prep_body,
        grid=(rows // BR,),
        in_specs=[
            pl.BlockSpec((BR, 64), lambda i: (i, 0)),
            pl.BlockSpec((BR, 64), lambda i: (i, 0)),
            pl.BlockSpec((BR, FPAD), lambda i: (i, 0)),
            pl.BlockSpec((64, 64), lambda i: (0, 0)),
            pl.BlockSpec((64, 64), lambda i: (0, 0)),
            pl.BlockSpec((1, 64), lambda i: (0, 0)),
        ],
        out_specs=[
            pl.BlockSpec((BR, 64), lambda i: (i, 0)),
            pl.BlockSpec((BR, TW), lambda i: (i, 0)),
        ],
        out_shape=[
            jax.ShapeDtypeStruct((rows, 64), jnp.float32),
            jax.ShapeDtypeStruct((rows, TW), jnp.float32),
        ],
    )(p1f, p2f, x2f, w1a, w1b, b1)


def _bf16(x):
    return x.astype(jnp.bfloat16).astype(jnp.float32)


def _rigid_body(g_ref, out_ref):
    g = g_ref[...]
    axx = ayy = azz = axy = axz = ayz = 0.0
    bx = by = bz = 0.0
    for k in range(MIN_COUNT):
        o = k * TW
        cx = g[:, o:o + 1]
        cy = g[:, o + 1:o + 2]
        cz = g[:, o + 2:o + 3]
        v = g[:, o + 3:o + 4]
        nrm = jnp.sqrt(cx * cx + cy * cy + cz * cz)
        ux, uy, uz = _bf16(cx / nrm), _bf16(cy / nrm), _bf16(cz / nrm)
        vb = _bf16(v)
        axx += ux * ux
        ayy += uy * uy
        azz += uz * uz
        axy += ux * uy
        axz += ux * uz
        ayz += uy * uz
        bx += ux * vb
        by += uy * vb
        bz += uz * vb
    axx += 1e-06
    ayy += 1e-06
    azz += 1e-06
    rows = [[axx, axy, axz, bx], [axy, ayy, ayz, by], [axz, ayz, azz, bz]]
    m0, m1, m2 = jnp.abs(axx), jnp.abs(axy), jnp.abs(axz)
    c0 = (m0 >= m1) & (m0 >= m2)
    c1 = (~c0) & (m1 >= m2)

    def pick(v):
        return jnp.where(c0, v[0], jnp.where(c1, v[1], v[2]))

    r0 = [pick([rows[0][j], rows[1][j], rows[2][j]]) for j in range(4)]
    r1 = [pick([rows[1][j], rows[0][j], rows[0][j]]) for j in range(4)]
    r2 = [pick([rows[2][j], rows[2][j], rows[1][j]]) for j in range(4)]
    l10 = r1[0] / r0[0]
    l20 = r2[0] / r0[0]
    r1 = [r1[j] - l10 * r0[j] for j in range(4)]
    r2 = [r2[j] - l20 * r0[j] for j in range(4)]
    swap = jnp.abs(r2[1]) > jnp.abs(r1[1])
    s1 = [jnp.where(swap, r2[j], r1[j]) for j in range(4)]
    s2 = [jnp.where(swap, r1[j], r2[j]) for j in range(4)]
    l21 = s2[1] / s1[1]
    s2 = [s2[j] - l21 * s1[j] for j in range(4)]
    rz = s2[3] / s2[2]
    ry = (s1[3] - s1[2] * rz) / s1[1]
    rx = (r0[3] - r0[1] * ry - r0[2] * rz) / r0[0]
    out_ref[...] = jnp.concatenate(
        [rx, ry, rz, jnp.zeros((rx.shape[0], FPAD - 3), jnp.float32)], axis=1)


def _rigid(g0):
    rows = g0.shape[0]
    return pl.pallas_call(
        _rigid_body,
        grid=(rows // BR,),
        in_specs=[pl.BlockSpec((BR, MIN_COUNT * TW), lambda i: (i, 0))],
        out_specs=pl.BlockSpec((BR, FPAD), lambda i: (i, 0)),
        out_shape=jax.ShapeDtypeStruct((rows, FPAD), jnp.float32),
    )(g0)


def _dot(a, b):
    return lax.dot_general(a, b, (((1,), (0,)), ((), ())),
                           preferred_element_type=jnp.float32,
                           precision=lax.Precision.DEFAULT)


def _leaky(x):
    return jnp.where(x >= 0, x, 0.1 * x)


def _wnet(d, wts):
    (v1, c1), (v2, c2), (v3, c3) = wts
    l1 = jnp.maximum(_dot(d, v1) + c1, 0.0)
    l2 = jnp.maximum(_dot(l1, v2) + c2, 0.0)
    return jnp.maximum(_dot(l2, v3) + c3, 0.0)


def _rep_rows(x, k):
    r, c = x.shape
    return jnp.reshape(jnp.broadcast_to(x[:, None, :], (r, k, c)), (r * k, c))


def _p2p_body(g_ref, a_ref, x1_ref, w1c_ref, w2_ref, b2_ref,
              v1_ref, c1_ref, v2_ref, c2_ref, v3_ref, c3_ref, out_ref):
    g = g_ref[...]
    a = _rep_rows(a_ref[...], NSAMPLE)
    x1 = _rep_rows(x1_ref[...][:, :3], NSAMPLE)
    d = g[:, 64:67] - x1
    pre = a + g[:, :64] + _dot(d, w1c_ref[...])
    h = _leaky(pre)
    h2 = _leaky(_dot(h, w2_ref[...]) + b2_ref[...])
    w = _wnet(d, [(v1_ref[...], c1_ref[...]), (v2_ref[...], c2_ref[...]),
                  (v3_ref[...], c3_ref[...])])
    prod = w * h2
    out_ref[...] = jnp.sum(
        jnp.reshape(prod, (BR, NSAMPLE, 64)), axis=1)


def _p2p_stage(g, a, x1f, w1c, w2, b2, wn):
    rows = a.shape[0]
    (v1, c1), (v2, c2), (v3, c3) = wn
    return pl.pallas_call(
        _p2p_body,
        grid=(rows // BR,),
        in_specs=[
            pl.BlockSpec((BR * NSAMPLE, TW), lambda i: (i, 0)),
            pl.BlockSpec((BR, 64), lambda i: (i, 0)),
            pl.BlockSpec((BR, FPAD), lambda i: (i, 0)),
            pl.BlockSpec((3, 64), lambda i: (0, 0)),
            pl.BlockSpec((64, 64), lambda i: (0, 0)),
            pl.BlockSpec((1, 64), lambda i: (0, 0)),
            pl.BlockSpec((3, 8), lambda i: (0, 0)),
            pl.BlockSpec((1, 8), lambda i: (0, 0)),
            pl.BlockSpec((8, 8), lambda i: (0, 0)),
            pl.BlockSpec((1, 8), lambda i: (0, 0)),
            pl.BlockSpec((8, 64), lambda i: (0, 0)),
            pl.BlockSpec((1, 64), lambda i: (0, 0)),
        ],
        out_specs=pl.BlockSpec((BR, 64), lambda i: (i, 0)),
        out_shape=jax.ShapeDtypeStruct((rows, 64), jnp.float32),
    )(g, a, x1f, w1c, w2, b2, v1, c1, v2, c2, v3, c3)


def _patch_body(g_ref, x1_ref, u1_ref, e1_ref, u2_ref, e2_ref,
                u3_ref, e3_ref, out_ref):
    g = g_ref[...]
    x1 = _rep_rows(x1_ref[...][:, :3], NSAMPLE)
    d = g[:, 64:67] - x1
    w = _wnet(d, [(u1_ref[...], e1_ref[...]), (u2_ref[...], e2_ref[...]),
                  (u3_ref[...], e3_ref[...])])
    prod = w * g[:, :64]
    out_ref[...] = jnp.sum(
        jnp.reshape(prod, (BR, NSAMPLE, 64)), axis=1)


def _patch_stage(g, x1f, wn):
    rows = x1f.shape[0]
    (u1, e1), (u2, e2), (u3, e3) = wn
    return pl.pallas_call(
        _patch_body,
        grid=(rows // BR,),
        in_specs=[
            pl.BlockSpec((BR * NSAMPLE, TW), lambda i: (i, 0)),
            pl.BlockSpec((BR, FPAD), lambda i: (i, 0)),
            pl.BlockSpec((3, 8), lambda i: (0, 0)),
            pl.BlockSpec((1, 8), lambda i: (0, 0)),
            pl.BlockSpec((8, 8), lambda i: (0, 0)),
            pl.BlockSpec((1, 8), lambda i: (0, 0)),
            pl.BlockSpec((8, 64), lambda i: (0, 0)),
            pl.BlockSpec((1, 64), lambda i: (0, 0)),
        ],
        out_specs=pl.BlockSpec((BR, 64), lambda i: (i, 0)),
        out_shape=jax.ShapeDtypeStruct((rows, 64), jnp.float32),
    )(g, x1f, u1, e1, u2, e2, u3, e3)


def _pad_last(x, width):
    return jnp.pad(x, [(0, 0)] * (x.ndim - 1) + [(0, width - x.shape[-1])])


def kernel(xyz1, xyz2, points1, points2, vel1, vel2, mask1, mask2, generator,
           w_xyz, w_vel, w_points, mlp_w0, mlp_b0, mlp_w1, mlp_b1,
           wn1_w0, wn1_b0, wn1_w1, wn1_b1, wn1_w2, wn1_b2,
           wn2_w0, wn2_b0, wn2_w1, wn2_b1, wn2_w2, wn2_b2):
    B = xyz1.shape[0]
    x1 = jnp.swapaxes(xyz1, 1, 2)
    x2 = jnp.swapaxes(xyz2, 1, 2)
    p1 = jnp.swapaxes(points1, 1, 2)
    p2 = jnp.swapaxes(points2, 1, 2)

    x1p = _pad_last(x1, FPAD)
    x2p = _pad_last(x2, FPAD)
    x1f = jnp.reshape(x1p, (B * N, FPAD))
    x2f = jnp.reshape(x2p, (B * N, FPAD))
    p1f = jnp.reshape(p1, (B * N, 64))
    p2f = jnp.reshape(p2, (B * N, 64))
    roff = (jnp.arange(B, dtype=jnp.int32) * N)[:, None, None]

    cidx = _knn_topk(x1p, x2p, MIN_COUNT)
    cflat = jnp.reshape(cidx + roff, (B * N * MIN_COUNT,))

    f1 = _pad_last(jnp.concatenate([w_xyz * x1, w_points * p1], axis=-1), FPAD)
    f2 = _pad_last(jnp.concatenate([w_xyz * x2, w_points * p2], axis=-1), FPAD)
    kidx = _knn_topk(f1, f2, NSAMPLE)
    kflat = jnp.reshape(kidx + roff, (B * N * NSAMPLE,))

    tab0 = _pad_last(
        jnp.concatenate([x2f[:, :3], jnp.reshape(vel2, (B * N, 1))], axis=1), TW)
    g0 = _sc_gather(tab0, cflat)
    rigidp = _rigid(jnp.reshape(g0, (B * N, MIN_COUNT * TW)))
    rigid = jnp.reshape(rigidp[:, :3], (B, N, 3))

    kidx2 = _knn_topk(jnp.reshape(rigidp, (B, N, FPAD)),
                      jnp.reshape(rigidp, (B, N, FPAD)), NSAMPLE)
    k2flat = jnp.reshape(kidx2 + roff, (B * N * NSAMPLE,))

    w1a = jnp.transpose(mlp_w0[:, :64])
    w1b = jnp.transpose(mlp_w0[:, 64:128])
    w1c = jnp.transpose(mlp_w0[:, 128:131])
    a, t = _prep(p1f, p2f, x2f, w1a, w1b, mlp_b0[None, :])

    g1 = _sc_gather(t, kflat)
    wn1 = [(jnp.transpose(wn1_w0), wn1_b0[None, :]),
           (jnp.transpose(wn1_w1), wn1_b1[None, :]),
           (jnp.transpose(wn1_w2), wn1_b2[None, :])]
    p2p = _p2p_stage(g1, a, x1f, w1c, jnp.transpose(mlp_w1),
                     mlp_b1[None, :], wn1)

    tab2 = _pad_last(jnp.concatenate([p2p, x1f[:, :3]], axis=1), TW)
    g2 = _sc_gather(tab2, k2flat)
    wn2 = [(jnp.transpose(wn2_w0), wn2_b0[None, :]),
           (jnp.transpose(wn2_w1), wn2_b1[None, :]),
           (jnp.transpose(wn2_w2), wn2_b2[None, :])]
    patch = _patch_stage(g2, x1f, wn2)

    patch = jnp.transpose(jnp.reshape(patch, (B, N, 64)), (0, 2, 1))
    return (patch, rigid)


# speedup vs baseline: 20.5013x; 1.0025x over previous
import functools

import jax
import jax.numpy as jnp
from jax import lax
from jax.experimental import pallas as pl
from jax.experimental.pallas import tpu as pltpu
from jax.experimental.pallas import tpu_sc as plsc

NSAMPLE = 16
MIN_COUNT = 8
N = 4096
BQ = 256
BR = 256
FPAD = 128
TW = 128


def _knn_body(k, q_ref, db_ref, out_ref):
    q = q_ref[0]
    db = db_ref[0]
    qn = jnp.sum(q * q, axis=1, keepdims=True)
    dn = jnp.sum(db * db, axis=1, keepdims=True).T
    d2 = lax.dot_general(q, db, (((1,), (1,)), ((), ())),
                         preferred_element_type=jnp.float32,
                         precision=lax.Precision.DEFAULT)
    nd = jnp.minimum((2.0 * d2 - qn) - dn, 0.0)
    iota = lax.broadcasted_iota(jnp.int32, (BQ, N), 1).astype(jnp.float32)
    cols = []
    idx = None
    for j in range(k):
        if j > 0:
            nd = jnp.where(iota == idx, -jnp.inf, nd)
        m = jnp.max(nd, axis=1, keepdims=True)
        sel = jnp.where(nd == m, iota, float(N))
        idx = jnp.min(sel, axis=1, keepdims=True)
        cols.append(idx)
    icols = jnp.concatenate(cols, axis=1).astype(jnp.int32)
    out_ref[0] = jnp.concatenate(
        [icols, jnp.zeros((BQ, FPAD - k), jnp.int32)], axis=1)


def _knn_topk(q, db, k):
    b = q.shape[0]
    out = pl.pallas_call(
        functools.partial(_knn_body, k),
        grid=(b, N // BQ),
        in_specs=[
            pl.BlockSpec((1, BQ, FPAD), lambda bi, i: (bi, i, 0)),
            pl.BlockSpec((1, N, FPAD), lambda bi, i: (bi, 0, 0)),
        ],
        out_specs=pl.BlockSpec((1, BQ, FPAD), lambda bi, i: (bi, i, 0)),
        out_shape=jax.ShapeDtypeStruct((b, N, FPAD), jnp.int32),
    )(q, db)
    return out[:, :, :k]


def _sc_gather(table, idx):
    _, d = table.shape
    (m,) = idx.shape
    info = plsc.get_sparse_core_info()
    nw = info.num_cores * info.num_subcores
    m_per_w = m // nw
    ch = min(m_per_w, 256)
    n_ch = m_per_w // ch
    mesh = plsc.VectorSubcoreMesh(core_axis_name="c", subcore_axis_name="s")

    @functools.partial(
        pl.kernel, mesh=mesh,
        out_type=jax.ShapeDtypeStruct((m, d), jnp.float32),
        scratch_types=[
            pltpu.VMEM((m_per_w,), jnp.int32),
            pltpu.VMEM((ch, d), jnp.float32),
            pltpu.VMEM((ch, d), jnp.float32),
            pltpu.SemaphoreType.DMA,
            pltpu.SemaphoreType.DMA,
            pltpu.SemaphoreType.DMA,
            pltpu.SemaphoreType.DMA,
        ],
    )
    def gk(table_hbm, idx_hbm, out_hbm, idx_v, rows0, rows1, g0, g1, w0, w1):
        wid = lax.axis_index("s") * info.num_cores + lax.axis_index("c")
        base = wid * m_per_w
        bufs, gsems, wsems = [rows0, rows1], [g0, g1], [w0, w1]
        pltpu.sync_copy(idx_hbm.at[pl.ds(base, m_per_w)], idx_v)

        def issue(c):
            return pltpu.async_copy(
                table_hbm.at[idx_v.at[pl.ds(c * ch, ch)]],
                bufs[c % 2], gsems[c % 2])

        gh = {0: issue(0)}
        if n_ch > 1:
            gh[1] = issue(1)
        wh = {}
        for c in range(n_ch):
            gh[c].wait()
            wh[c] = pltpu.async_copy(
                bufs[c % 2], out_hbm.at[pl.ds(base + c * ch, ch)],
                wsems[c % 2])
            if c + 2 < n_ch:
                wh[c].wait()
                gh[c + 2] = issue(c + 2)
        for c in (n_ch - 2, n_ch - 1):
            if c >= 0 and c in wh and c + 2 >= n_ch:
                wh[c].wait()

    return gk(table, idx)


def _prep_body(p1_ref, p2_ref, x2_ref, w1a_ref, w1b_ref, b1_ref,
               a_ref, t_ref):
    p1 = p1_ref[...]
    p2 = p2_ref[...]
    a_ref[...] = lax.dot_general(p1, w1a_ref[...], (((1,), (0,)), ((), ())),
                                 preferred_element_type=jnp.float32) + b1_ref[...]
    t64 = lax.dot_general(p2, w1b_ref[...], (((1,), (0,)), ((), ())),
                          preferred_element_type=jnp.float32)
    x2 = x2_ref[...][:, :3]
    t_ref[...] = jnp.concatenate(
        [t64, x2, jnp.zeros((t64.shape[0], TW - 67), jnp.float32)], axis=1)


def _prep(p1f, p2f, x2f, w1a, w1b, b1):
    rows = p1f.shape[0]
    return pl.pallas_call(
        _prep_body,
        grid=(rows // BR,),
        in_specs=[
            pl.BlockSpec((BR, 64), lambda i: (i, 0)),
            pl.BlockSpec((BR, 64), lambda i: (i, 0)),
            pl.BlockSpec((BR, FPAD), lambda i: (i, 0)),
            pl.BlockSpec((64, 64), lambda i: (0, 0)),
            pl.BlockSpec((64, 64), lambda i: (0, 0)),
            pl.BlockSpec((1, 64), lambda i: (0, 0)),
        ],
        out_specs=[
            pl.BlockSpec((BR, 64), lambda i: (i, 0)),
            pl.BlockSpec((BR, TW), lambda i: (i, 0)),
        ],
        out_shape=[
            jax.ShapeDtypeStruct((rows, 64), jnp.float32),
            jax.ShapeDtypeStruct((rows, TW), jnp.float32),
        ],
    )(p1f, p2f, x2f, w1a, w1b, b1)


def _bf16(x):
    return x.astype(jnp.bfloat16).astype(jnp.float32)


def _rigid_body(g_ref, out_ref):
    g = g_ref[...]
    axx = ayy = azz = axy = axz = ayz = 0.0
    bx = by = bz = 0.0
    for k in range(MIN_COUNT):
        o = k * TW
        cx = g[:, o:o + 1]
        cy = g[:, o + 1:o + 2]
        cz = g[:, o + 2:o + 3]
        v = g[:, o + 3:o + 4]
        nrm = jnp.sqrt(cx * cx + cy * cy + cz * cz)
        ux, uy, uz = _bf16(cx / nrm), _bf16(cy / nrm), _bf16(cz / nrm)
        vb = _bf16(v)
        axx += ux * ux
        ayy += uy * uy
        azz += uz * uz
        axy += ux * uy
        axz += ux * uz
        ayz += uy * uz
        bx += ux * vb
        by += uy * vb
        bz += uz * vb
    axx += 1e-06
    ayy += 1e-06
    azz += 1e-06
    rows = [[axx, axy, axz, bx], [axy, ayy, ayz, by], [axz, ayz, azz, bz]]
    m0, m1, m2 = jnp.abs(axx), jnp.abs(axy), jnp.abs(axz)
    c0 = (m0 >= m1) & (m0 >= m2)
    c1 = (~c0) & (m1 >= m2)

    def pick(v):
        return jnp.where(c0, v[0], jnp.where(c1, v[1], v[2]))

    r0 = [pick([rows[0][j], rows[1][j], rows[2][j]]) for j in range(4)]
    r1 = [pick([rows[1][j], rows[0][j], rows[0][j]]) for j in range(4)]
    r2 = [pick([rows[2][j], rows[2][j], rows[1][j]]) for j in range(4)]
    l10 = r1[0] / r0[0]
    l20 = r2[0] / r0[0]
    r1 = [r1[j] - l10 * r0[j] for j in range(4)]
    r2 = [r2[j] - l20 * r0[j] for j in range(4)]
    swap = jnp.abs(r2[1]) > jnp.abs(r1[1])
    s1 = [jnp.where(swap, r2[j], r1[j]) for j in range(4)]
    s2 = [jnp.where(swap, r1[j], r2[j]) for j in range(4)]
    l21 = s2[1] / s1[1]
    s2 = [s2[j] - l21 * s1[j] for j in range(4)]
    rz = s2[3] / s2[2]
    ry = (s1[3] - s1[2] * rz) / s1[1]
    rx = (r0[3] - r0[1] * ry - r0[2] * rz) / r0[0]
    out_ref[...] = jnp.concatenate(
        [rx, ry, rz, jnp.zeros((rx.shape[0], FPAD - 3), jnp.float32)], axis=1)


def _rigid(g0):
    rows = g0.shape[0]
    return pl.pallas_call(
        _rigid_body,
        grid=(rows // BR,),
        in_specs=[pl.BlockSpec((BR, MIN_COUNT * TW), lambda i: (i, 0))],
        out_specs=pl.BlockSpec((BR, FPAD), lambda i: (i, 0)),
        out_shape=jax.ShapeDtypeStruct((rows, FPAD), jnp.float32),
    )(g0)


def _dot(a, b):
    return lax.dot_general(a, b, (((1,), (0,)), ((), ())),
                           preferred_element_type=jnp.float32,
                           precision=lax.Precision.DEFAULT)


def _leaky(x):
    return jnp.where(x >= 0, x, 0.1 * x)


def _wnet(d, wts):
    (v1, c1), (v2, c2), (v3, c3) = wts
    l1 = jnp.maximum(_dot(d, v1) + c1, 0.0)
    l2 = jnp.maximum(_dot(l1, v2) + c2, 0.0)
    return jnp.maximum(_dot(l2, v3) + c3, 0.0)


def _rep_rows(x, k):
    r, c = x.shape
    return jnp.reshape(jnp.broadcast_to(x[:, None, :], (r, k, c)), (r * k, c))


def _p2p_body(g_ref, a_ref, x1_ref, w1c_ref, w2_ref, b2_ref,
              v1_ref, c1_ref, v2_ref, c2_ref, v3_ref, c3_ref, out_ref):
    g = g_ref[...]
    a = _rep_rows(a_ref[...], NSAMPLE)
    x1 = _rep_rows(x1_ref[...][:, :3], NSAMPLE)
    d = g[:, 64:67] - x1
    pre = a + g[:, :64] + _dot(d, w1c_ref[...])
    h = _leaky(pre)
    h2 = _leaky(_dot(h, w2_ref[...]) + b2_ref[...])
    w = _wnet(d, [(v1_ref[...], c1_ref[...]), (v2_ref[...], c2_ref[...]),
                  (v3_ref[...], c3_ref[...])])
    prod = w * h2
    out_ref[...] = jnp.sum(
        jnp.reshape(prod, (BR, NSAMPLE, 64)), axis=1)


def _p2p_stage(g, a, x1f, w1c, w2, b2, wn):
    rows = a.shape[0]
    (v1, c1), (v2, c2), (v3, c3) = wn
    return pl.pallas_call(
        _p2p_body,
        grid=(rows // BR,),
        in_specs=[
            pl.BlockSpec((BR * NSAMPLE, TW), lambda i: (i, 0)),
            pl.BlockSpec((BR, 64), lambda i: (i, 0)),
            pl.BlockSpec((BR, FPAD), lambda i: (i, 0)),
            pl.BlockSpec((3, 64), lambda i: (0, 0)),
            pl.BlockSpec((64, 64), lambda i: (0, 0)),
            pl.BlockSpec((1, 64), lambda i: (0, 0)),
            pl.BlockSpec((3, 8), lambda i: (0, 0)),
            pl.BlockSpec((1, 8), lambda i: (0, 0)),
            pl.BlockSpec((8, 8), lambda i: (0, 0)),
            pl.BlockSpec((1, 8), lambda i: (0, 0)),
            pl.BlockSpec((8, 64), lambda i: (0, 0)),
            pl.BlockSpec((1, 64), lambda i: (0, 0)),
        ],
        out_specs=pl.BlockSpec((BR, 64), lambda i: (i, 0)),
        out_shape=jax.ShapeDtypeStruct((rows, 64), jnp.float32),
    )(g, a, x1f, w1c, w2, b2, v1, c1, v2, c2, v3, c3)


def _patch_body(g_ref, x1_ref, u1_ref, e1_ref, u2_ref, e2_ref,
                u3_ref, e3_ref, out_ref):
    g = g_ref[...]
    x1 = _rep_rows(x1_ref[...][:, :3], NSAMPLE)
    d = g[:, 64:67] - x1
    w = _wnet(d, [(u1_ref[...], e1_ref[...]), (u2_ref[...], e2_ref[...]),
                  (u3_ref[...], e3_ref[...])])
    prod = w * g[:, :64]
    out_ref[...] = jnp.sum(
        jnp.reshape(prod, (BR, NSAMPLE, 64)), axis=1)


def _patch_stage(g, x1f, wn):
    rows = x1f.shape[0]
    (u1, e1), (u2, e2), (u3, e3) = wn
    return pl.pallas_call(
        _patch_body,
        grid=(rows // BR,),
        in_specs=[
            pl.BlockSpec((BR * NSAMPLE, TW), lambda i: (i, 0)),
            pl.BlockSpec((BR, FPAD), lambda i: (i, 0)),
            pl.BlockSpec((3, 8), lambda i: (0, 0)),
            pl.BlockSpec((1, 8), lambda i: (0, 0)),
            pl.BlockSpec((8, 8), lambda i: (0, 0)),
            pl.BlockSpec((1, 8), lambda i: (0, 0)),
            pl.BlockSpec((8, 64), lambda i: (0, 0)),
            pl.BlockSpec((1, 64), lambda i: (0, 0)),
        ],
        out_specs=pl.BlockSpec((BR, 64), lambda i: (i, 0)),
        out_shape=jax.ShapeDtypeStruct((rows, 64), jnp.float32),
    )(g, x1f, u1, e1, u2, e2, u3, e3)


def _pad_last(x, width):
    return jnp.pad(x, [(0, 0)] * (x.ndim - 1) + [(0, width - x.shape[-1])])


def kernel(xyz1, xyz2, points1, points2, vel1, vel2, mask1, mask2, generator,
           w_xyz, w_vel, w_points, mlp_w0, mlp_b0, mlp_w1, mlp_b1,
           wn1_w0, wn1_b0, wn1_w1, wn1_b1, wn1_w2, wn1_b2,
           wn2_w0, wn2_b0, wn2_w1, wn2_b1, wn2_w2, wn2_b2):
    B = xyz1.shape[0]
    x1 = jnp.swapaxes(xyz1, 1, 2)
    x2 = jnp.swapaxes(xyz2, 1, 2)
    p1 = jnp.swapaxes(points1, 1, 2)
    p2 = jnp.swapaxes(points2, 1, 2)

    x1p = _pad_last(x1, FPAD)
    x2p = _pad_last(x2, FPAD)
    x1f = jnp.reshape(x1p, (B * N, FPAD))
    x2f = jnp.reshape(x2p, (B * N, FPAD))
    p1f = jnp.reshape(p1, (B * N, 64))
    p2f = jnp.reshape(p2, (B * N, 64))
    roff = (jnp.arange(B, dtype=jnp.int32) * N)[:, None, None]

    cidx = _knn_topk(x1p, x2p, MIN_COUNT)
    cflat = jnp.reshape(cidx + roff, (B * N * MIN_COUNT,))

    f1 = _pad_last(jnp.concatenate([w_xyz * x1, w_points * p1], axis=-1), FPAD)
    f2 = _pad_last(jnp.concatenate([w_xyz * x2, w_points * p2], axis=-1), FPAD)
    kidx = _knn_topk(f1, f2, NSAMPLE)
    kflat = jnp.reshape(kidx + roff, (B * N * NSAMPLE,))

    tab0 = _pad_last(
        jnp.concatenate([x2f[:, :3], jnp.reshape(vel2, (B * N, 1))], axis=1), TW)
    g0 = _sc_gather(tab0, cflat)
    rigidp = _rigid(jnp.reshape(g0, (B * N, MIN_COUNT * TW)))
    rigid = jnp.reshape(rigidp[:, :3], (B, N, 3))

    kidx2 = _knn_topk(jnp.reshape(rigidp, (B, N, FPAD)),
                      jnp.reshape(rigidp, (B, N, FPAD)), NSAMPLE)
    k2flat = jnp.reshape(kidx2 + roff, (B * N * NSAMPLE,))

    w1a = jnp.transpose(mlp_w0[:, :64])
    w1b = jnp.transpose(mlp_w0[:, 64:128])
    w1c = jnp.transpose(mlp_w0[:, 128:131])
    a, t = _prep(p1f, p2f, x2f, w1a, w1b, mlp_b0[None, :])

    g1 = _sc_gather(t, kflat)
    wn1 = [(jnp.transpose(wn1_w0), wn1_b0[None, :]),
           (jnp.transpose(wn1_w1), wn1_b1[None, :]),
           (jnp.transpose(wn1_w2), wn1_b2[None, :])]
    p2p = _p2p_stage(g1, a, x1f, w1c, jnp.transpose(mlp_w1),
                     mlp_b1[None, :], wn1)

    tab2 = _pad_last(jnp.concatenate([p2p, x1f[:, :3]], axis=1), TW)
    g2 = _sc_gather(tab2, k2flat)
    wn2 = [(jnp.transpose(wn2_w0), wn2_b0[None, :]),
           (jnp.transpose(wn2_w1), wn2_b1[None, :]),
           (jnp.transpose(wn2_w2), wn2_b2[None, :])]
    patch = _patch_stage(g2, x1f, wn2)

    patch = jnp.transpose(jnp.reshape(patch, (B, N, 64)), (0, 2, 1))
    return (patch, rigid)


# speedup vs baseline: 21.5211x; 1.0497x over previous
import functools

import jax
import jax.numpy as jnp
from jax import lax
from jax.experimental import pallas as pl
from jax.experimental.pallas import tpu as pltpu
from jax.experimental.pallas import tpu_sc as plsc

NSAMPLE = 16
MIN_COUNT = 8
N = 4096
BQ = 256
BR = 256
FPAD = 128
TW = 128


def _topk_from_nd(nd, k, out_ref):
    iota = lax.broadcasted_iota(jnp.int32, nd.shape, 1).astype(jnp.float32)
    cols = []
    idx = None
    for j in range(k):
        if j > 0:
            nd = jnp.where(iota == idx, -jnp.inf, nd)
        m = jnp.max(nd, axis=1, keepdims=True)
        sel = jnp.where(nd == m, iota, float(N))
        idx = jnp.min(sel, axis=1, keepdims=True)
        cols.append(idx)
    icols = jnp.concatenate(cols, axis=1).astype(jnp.int32)
    out_ref[0] = jnp.concatenate(
        [icols, jnp.zeros((nd.shape[0], FPAD - k), jnp.int32)], axis=1)


def _nd_from_qdb(q, db):
    qn = jnp.transpose(jnp.sum(q * q, axis=0, keepdims=True))
    dn = jnp.sum(db * db, axis=0, keepdims=True)
    d2 = lax.dot_general(q, db, (((0,), (0,)), ((), ())),
                         preferred_element_type=jnp.float32,
                         precision=lax.Precision.DEFAULT)
    return jnp.minimum((2.0 * d2 - qn) - dn, 0.0)


def _knn_xyz_body(k, q_ref, db_ref, out_ref):
    _topk_from_nd(_nd_from_qdb(q_ref[0], db_ref[0]), k, out_ref)


def _knn_xyz(xyz_q, xyz_db, k):
    b = xyz_q.shape[0]
    out = pl.pallas_call(
        functools.partial(_knn_xyz_body, k),
        grid=(b, N // BQ),
        in_specs=[
            pl.BlockSpec((1, 3, BQ), lambda bi, i: (bi, 0, i)),
            pl.BlockSpec((1, 3, N), lambda bi, i: (bi, 0, 0)),
        ],
        out_specs=pl.BlockSpec((1, BQ, FPAD), lambda bi, i: (bi, i, 0)),
        out_shape=jax.ShapeDtypeStruct((b, N, FPAD), jnp.int32),
    )(xyz_q, xyz_db)
    return out[:, :, :k]


def _knn_feat_body(k, wx_ref, wp_ref, xq_ref, pq_ref, xdb_ref, pdb_ref,
                   out_ref):
    wx = wx_ref[0, 0]
    wp = wp_ref[0, 0]
    q = jnp.concatenate([wx * xq_ref[0], wp * pq_ref[0]], axis=0)
    db = jnp.concatenate([wx * xdb_ref[0], wp * pdb_ref[0]], axis=0)
    _topk_from_nd(_nd_from_qdb(q, db), k, out_ref)


def _knn_feat(wx, wp, xyz_q, p_q, xyz_db, p_db, k):
    b = xyz_q.shape[0]
    out = pl.pallas_call(
        functools.partial(_knn_feat_body, k),
        grid=(b, N // BQ),
        in_specs=[
            pl.BlockSpec(memory_space=pltpu.SMEM),
            pl.BlockSpec(memory_space=pltpu.SMEM),
            pl.BlockSpec((1, 3, BQ), lambda bi, i: (bi, 0, i)),
            pl.BlockSpec((1, 64, BQ), lambda bi, i: (bi, 0, i)),
            pl.BlockSpec((1, 3, N), lambda bi, i: (bi, 0, 0)),
            pl.BlockSpec((1, 64, N), lambda bi, i: (bi, 0, 0)),
        ],
        out_specs=pl.BlockSpec((1, BQ, FPAD), lambda bi, i: (bi, i, 0)),
        out_shape=jax.ShapeDtypeStruct((b, N, FPAD), jnp.int32),
    )(wx.reshape(1, 1), wp.reshape(1, 1), xyz_q, p_q, xyz_db, p_db)
    return out[:, :, :k]


def _knn_rigid_body(k, q_ref, db_ref, out_ref):
    q = q_ref[0]
    db = db_ref[0]
    qn = jnp.sum(q * q, axis=1, keepdims=True)
    dn = jnp.transpose(jnp.sum(db * db, axis=1, keepdims=True))
    d2 = lax.dot_general(q, db, (((1,), (1,)), ((), ())),
                         preferred_element_type=jnp.float32,
                         precision=lax.Precision.DEFAULT)
    nd = jnp.minimum((2.0 * d2 - qn) - dn, 0.0)
    _topk_from_nd(nd, k, out_ref)


def _knn_rigid(rp, k):
    b = rp.shape[0]
    out = pl.pallas_call(
        functools.partial(_knn_rigid_body, k),
        grid=(b, N // BQ),
        in_specs=[
            pl.BlockSpec((1, BQ, FPAD), lambda bi, i: (bi, i, 0)),
            pl.BlockSpec((1, N, FPAD), lambda bi, i: (bi, 0, 0)),
        ],
        out_specs=pl.BlockSpec((1, BQ, FPAD), lambda bi, i: (bi, i, 0)),
        out_shape=jax.ShapeDtypeStruct((b, N, FPAD), jnp.int32),
    )(rp, rp)
    return out[:, :, :k]


def _sc_gather(table, idx):
    _, d = table.shape
    (m,) = idx.shape
    info = plsc.get_sparse_core_info()
    nw = info.num_cores * info.num_subcores
    m_per_w = m // nw
    ch = min(m_per_w, 256)
    n_ch = m_per_w // ch
    mesh = plsc.VectorSubcoreMesh(core_axis_name="c", subcore_axis_name="s")

    @functools.partial(
        pl.kernel, mesh=mesh,
        out_type=jax.ShapeDtypeStruct((m, d), jnp.float32),
        scratch_types=[
            pltpu.VMEM((m_per_w,), jnp.int32),
            pltpu.VMEM((ch, d), jnp.float32),
            pltpu.VMEM((ch, d), jnp.float32),
            pltpu.SemaphoreType.DMA,
            pltpu.SemaphoreType.DMA,
            pltpu.SemaphoreType.DMA,
            pltpu.SemaphoreType.DMA,
        ],
    )
    def gk(table_hbm, idx_hbm, out_hbm, idx_v, rows0, rows1, g0, g1, w0, w1):
        wid = lax.axis_index("s") * info.num_cores + lax.axis_index("c")
        base = wid * m_per_w
        bufs, gsems, wsems = [rows0, rows1], [g0, g1], [w0, w1]
        pltpu.sync_copy(idx_hbm.at[pl.ds(base, m_per_w)], idx_v)

        def issue(c):
            return pltpu.async_copy(
                table_hbm.at[idx_v.at[pl.ds(c * ch, ch)]],
                bufs[c % 2], gsems[c % 2])

        gh = {0: issue(0)}
        if n_ch > 1:
            gh[1] = issue(1)
        wh = {}
        for c in range(n_ch):
            gh[c].wait()
            wh[c] = pltpu.async_copy(
                bufs[c % 2], out_hbm.at[pl.ds(base + c * ch, ch)],
                wsems[c % 2])
            if c + 2 < n_ch:
                wh[c].wait()
                gh[c + 2] = issue(c + 2)
        for c in (n_ch - 2, n_ch - 1):
            if c >= 0 and c + 2 >= n_ch:
                wh[c].wait()

    return gk(table, idx)


def _prep_body(p1_ref, p2_ref, x2_ref, v2_ref, w1a_ref, w1b_ref, b1_ref,
               a_ref, t_ref, tab0_ref):
    p1 = p1_ref[0]
    p2 = p2_ref[0]
    a_ref[...] = lax.dot_general(p1, w1a_ref[...], (((0,), (0,)), ((), ())),
                                 preferred_element_type=jnp.float32) + b1_ref[...]
    t64 = lax.dot_general(p2, w1b_ref[...], (((0,), (0,)), ((), ())),
                          preferred_element_type=jnp.float32)
    x2r = jnp.transpose(x2_ref[0])
    v2r = jnp.transpose(v2_ref[0])
    zer = jnp.zeros((t64.shape[0], TW - 67), jnp.float32)
    t_ref[...] = jnp.concatenate([t64, x2r, zer], axis=1)
    tab0_ref[...] = jnp.concatenate(
        [x2r, v2r, jnp.zeros((t64.shape[0], TW - 4), jnp.float32)], axis=1)


def _prep(points1, points2, xyz2, vel2, w1a, w1b, b1):
    b = points1.shape[0]
    nb = N // BR
    return pl.pallas_call(
        _prep_body,
        grid=(b, nb),
        in_specs=[
            pl.BlockSpec((1, 64, BR), lambda bi, i: (bi, 0, i)),
            pl.BlockSpec((1, 64, BR), lambda bi, i: (bi, 0, i)),
            pl.BlockSpec((1, 3, BR), lambda bi, i: (bi, 0, i)),
            pl.BlockSpec((1, 1, BR), lambda bi, i: (bi, 0, i)),
            pl.BlockSpec((64, 64), lambda bi, i: (0, 0)),
            pl.BlockSpec((64, 64), lambda bi, i: (0, 0)),
            pl.BlockSpec((1, 64), lambda bi, i: (0, 0)),
        ],
        out_specs=[
            pl.BlockSpec((BR, 64), lambda bi, i: (bi * nb + i, 0)),
            pl.BlockSpec((BR, TW), lambda bi, i: (bi * nb + i, 0)),
            pl.BlockSpec((BR, TW), lambda bi, i: (bi * nb + i, 0)),
        ],
        out_shape=[
            jax.ShapeDtypeStruct((b * N, 64), jnp.float32),
            jax.ShapeDtypeStruct((b * N, TW), jnp.float32),
            jax.ShapeDtypeStruct((b * N, TW), jnp.float32),
        ],
    )(points1, points2, xyz2, vel2[:, None, :], w1a, w1b, b1)


def _bf16(x):
    return x.astype(jnp.bfloat16).astype(jnp.float32)


def _rigid_body(g_ref, out_ref):
    g = g_ref[...]
    axx = ayy = azz = axy = axz = ayz = 0.0
    bx = by = bz = 0.0
    for k in range(MIN_COUNT):
        o = k * TW
        cx = g[:, o:o + 1]
        cy = g[:, o + 1:o + 2]
        cz = g[:, o + 2:o + 3]
        v = g[:, o + 3:o + 4]
        nrm = jnp.sqrt(cx * cx + cy * cy + cz * cz)
        ux, uy, uz = _bf16(cx / nrm), _bf16(cy / nrm), _bf16(cz / nrm)
        vb = _bf16(v)
        axx += ux * ux
        ayy += uy * uy
        azz += uz * uz
        axy += ux * uy
        axz += ux * uz
        ayz += uy * uz
        bx += ux * vb
        by += uy * vb
        bz += uz * vb
    axx += 1e-06
    ayy += 1e-06
    azz += 1e-06
    rows = [[axx, axy, axz, bx], [axy, ayy, ayz, by], [axz, ayz, azz, bz]]
    m0, m1, m2 = jnp.abs(axx), jnp.abs(axy), jnp.abs(axz)
    c0 = (m0 >= m1) & (m0 >= m2)
    c1 = (~c0) & (m1 >= m2)

    def pick(v3):
        return jnp.where(c0, v3[0], jnp.where(c1, v3[1], v3[2]))

    r0 = [pick([rows[0][j], rows[1][j], rows[2][j]]) for j in range(4)]
    r1 = [pick([rows[1][j], rows[0][j], rows[0][j]]) for j in range(4)]
    r2 = [pick([rows[2][j], rows[2][j], rows[1][j]]) for j in range(4)]
    l10 = r1[0] / r0[0]
    l20 = r2[0] / r0[0]
    r1 = [r1[j] - l10 * r0[j] for j in range(4)]
    r2 = [r2[j] - l20 * r0[j] for j in range(4)]
    swap = jnp.abs(r2[1]) > jnp.abs(r1[1])
    s1 = [jnp.where(swap, r2[j], r1[j]) for j in range(4)]
    s2 = [jnp.where(swap, r1[j], r2[j]) for j in range(4)]
    l21 = s2[1] / s1[1]
    s2 = [s2[j] - l21 * s1[j] for j in range(4)]
    rz = s2[3] / s2[2]
    ry = (s1[3] - s1[2] * rz) / s1[1]
    rx = (r0[3] - r0[1] * ry - r0[2] * rz) / r0[0]
    out_ref[...] = jnp.concatenate(
        [rx, ry, rz, jnp.zeros((rx.shape[0], FPAD - 3), jnp.float32)], axis=1)


def _rigid(g0):
    rows = g0.shape[0]
    return pl.pallas_call(
        _rigid_body,
        grid=(rows // BR,),
        in_specs=[pl.BlockSpec((BR, MIN_COUNT * TW), lambda i: (i, 0))],
        out_specs=pl.BlockSpec((BR, FPAD), lambda i: (i, 0)),
        out_shape=jax.ShapeDtypeStruct((rows, FPAD), jnp.float32),
    )(g0)


def _dot(a, b):
    return lax.dot_general(a, b, (((1,), (0,)), ((), ())),
                           preferred_element_type=jnp.float32,
                           precision=lax.Precision.DEFAULT)


def _leaky(x):
    return jnp.where(x >= 0, x, 0.1 * x)


def _wnet(d, wts):
    (v1, c1), (v2, c2), (v3, c3) = wts
    l1 = jnp.maximum(_dot(d, v1) + c1, 0.0)
    l2 = jnp.maximum(_dot(l1, v2) + c2, 0.0)
    return jnp.maximum(_dot(l2, v3) + c3, 0.0)


def _rep_rows(x, k):
    r, c = x.shape
    return jnp.reshape(jnp.broadcast_to(x[:, None, :], (r, k, c)), (r * k, c))


def _p2p_body(g_ref, a_ref, x1_ref, w1c_ref, w2_ref, b2_ref,
              v1_ref, c1_ref, v2_ref, c2_ref, v3_ref, c3_ref, tab2_ref):
    g = g_ref[...]
    a = _rep_rows(a_ref[...], NSAMPLE)
    x1r = jnp.transpose(x1_ref[0])
    x1 = _rep_rows(x1r, NSAMPLE)
    d = g[:, 64:67] - x1
    pre = a + g[:, :64] + _dot(d, w1c_ref[...])
    h = _leaky(pre)
    h2 = _leaky(_dot(h, w2_ref[...]) + b2_ref[...])
    w = _wnet(d, [(v1_ref[...], c1_ref[...]), (v2_ref[...], c2_ref[...]),
                  (v3_ref[...], c3_ref[...])])
    p2p = jnp.sum(jnp.reshape(w * h2, (BR, NSAMPLE, 64)), axis=1)
    tab2_ref[...] = jnp.concatenate(
        [p2p, x1r, jnp.zeros((BR, TW - 67), jnp.float32)], axis=1)


def _p2p_stage(g, a, xyz1, w1c, w2, b2, wn):
    b = xyz1.shape[0]
    nb = N // BR
    (v1, c1), (v2, c2), (v3, c3) = wn
    return pl.pallas_call(
        _p2p_body,
        grid=(b, nb),
        in_specs=[
            pl.BlockSpec((BR * NSAMPLE, TW), lambda bi, i: (bi * nb + i, 0)),
            pl.BlockSpec((BR, 64), lambda bi, i: (bi * nb + i, 0)),
            pl.BlockSpec((1, 3, BR), lambda bi, i: (bi, 0, i)),
            pl.BlockSpec((3, 64), lambda bi, i: (0, 0)),
            pl.BlockSpec((64, 64), lambda bi, i: (0, 0)),
            pl.BlockSpec((1, 64), lambda bi, i: (0, 0)),
            pl.BlockSpec((3, 8), lambda bi, i: (0, 0)),
            pl.BlockSpec((1, 8), lambda bi, i: (0, 0)),
            pl.BlockSpec((8, 8), lambda bi, i: (0, 0)),
            pl.BlockSpec((1, 8), lambda bi, i: (0, 0)),
            pl.BlockSpec((8, 64), lambda bi, i: (0, 0)),
            pl.BlockSpec((1, 64), lambda bi, i: (0, 0)),
        ],
        out_specs=pl.BlockSpec((BR, TW), lambda bi, i: (bi * nb + i, 0)),
        out_shape=jax.ShapeDtypeStruct((b * N, TW), jnp.float32),
    )(g, a, xyz1, w1c, w2, b2, v1, c1, v2, c2, v3, c3)


def _patch_body(g_ref, x1_ref, u1_ref, e1_ref, u2_ref, e2_ref,
                u3_ref, e3_ref, out_ref):
    g = g_ref[...]
    x1 = _rep_rows(jnp.transpose(x1_ref[0]), NSAMPLE)
    d = g[:, 64:67] - x1
    w = _wnet(d, [(u1_ref[...], e1_ref[...]), (u2_ref[...], e2_ref[...]),
                  (u3_ref[...], e3_ref[...])])
    res = jnp.sum(jnp.reshape(w * g[:, :64], (BR, NSAMPLE, 64)), axis=1)
    out_ref[0] = jnp.transpose(res)


def _patch_stage(g, xyz1, wn):
    b = xyz1.shape[0]
    nb = N // BR
    (u1, e1), (u2, e2), (u3, e3) = wn
    return pl.pallas_call(
        _patch_body,
        grid=(b, nb),
        in_specs=[
            pl.BlockSpec((BR * NSAMPLE, TW), lambda bi, i: (bi * nb + i, 0)),
            pl.BlockSpec((1, 3, BR), lambda bi, i: (bi, 0, i)),
            pl.BlockSpec((3, 8), lambda bi, i: (0, 0)),
            pl.BlockSpec((1, 8), lambda bi, i: (0, 0)),
            pl.BlockSpec((8, 8), lambda bi, i: (0, 0)),
            pl.BlockSpec((1, 8), lambda bi, i: (0, 0)),
            pl.BlockSpec((8, 64), lambda bi, i: (0, 0)),
            pl.BlockSpec((1, 64), lambda bi, i: (0, 0)),
        ],
        out_specs=pl.BlockSpec((1, 64, BR), lambda bi, i: (bi, 0, i)),
        out_shape=jax.ShapeDtypeStruct((b, 64, N), jnp.float32),
    )(g, xyz1, u1, e1, u2, e2, u3, e3)


def kernel(xyz1, xyz2, points1, points2, vel1, vel2, mask1, mask2, generator,
           w_xyz, w_vel, w_points, mlp_w0, mlp_b0, mlp_w1, mlp_b1,
           wn1_w0, wn1_b0, wn1_w1, wn1_b1, wn1_w2, wn1_b2,
           wn2_w0, wn2_b0, wn2_w1, wn2_b1, wn2_w2, wn2_b2):
    B = xyz1.shape[0]
    roff = (jnp.arange(B, dtype=jnp.int32) * N)[:, None, None]

    cidx = _knn_xyz(xyz1, xyz2, MIN_COUNT)
    cflat = jnp.reshape(cidx + roff, (B * N * MIN_COUNT,))

    kidx = _knn_feat(w_xyz, w_points, xyz1, points1, xyz2, points2, NSAMPLE)
    kflat = jnp.reshape(kidx + roff, (B * N * NSAMPLE,))

    w1a = jnp.transpose(mlp_w0[:, :64])
    w1b = jnp.transpose(mlp_w0[:, 64:128])
    w1c = jnp.transpose(mlp_w0[:, 128:131])
    a, t, tab0 = _prep(points1, points2, xyz2, vel2, w1a, w1b, mlp_b0[None, :])

    g0 = _sc_gather(tab0, cflat)
    rigidp = _rigid(jnp.reshape(g0, (B * N, MIN_COUNT * TW)))
    rigid = jnp.reshape(rigidp[:, :3], (B, N, 3))

    kidx2 = _knn_rigid(jnp.reshape(rigidp, (B, N, FPAD)), NSAMPLE)
    k2flat = jnp.reshape(kidx2 + roff, (B * N * NSAMPLE,))

    g1 = _sc_gather(t, kflat)
    wn1 = [(jnp.transpose(wn1_w0), wn1_b0[None, :]),
           (jnp.transpose(wn1_w1), wn1_b1[None, :]),
           (jnp.transpose(wn1_w2), wn1_b2[None, :])]
    tab2 = _p2p_stage(g1, a, xyz1, w1c, jnp.transpose(mlp_w1),
                      mlp_b1[None, :], wn1)

    g2 = _sc_gather(tab2, k2flat)
    wn2 = [(jnp.transpose(wn2_w0), wn2_b0[None, :]),
           (jnp.transpose(wn2_w1), wn2_b1[None, :]),
           (jnp.transpose(wn2_w2), wn2_b2[None, :])]
    patch = _patch_stage(g2, xyz1, wn2)

    return (patch, rigid)
